# Initial kernel scaffold; baseline (speedup 1.0000x reference)
import jax
import jax.numpy as jnp
from jax.experimental import pallas as pl


def kernel(x, row_index, col_index, att_bias, Wq, bq, Wk, bk, Wv, bv, Wo, bo):
    raise NotImplementedError("write your pallas kernel here")


# speedup vs baseline: 6.5061x; 6.5061x over previous
import functools
import math

import jax
import jax.numpy as jnp
from jax.experimental import pallas as pl


def _qkv_proj_body(x_ref, w_ref, b_ref, out_ref):
    out_ref[...] = (
        jax.lax.dot_general(
            x_ref[...], w_ref[...], (((1,), (0,)), ((), ())),
            preferred_element_type=jnp.float32,
        )
        + b_ref[...]
    )


def _attn_body(q_ref, k_ref, v_ref, m_ref, out_ref):
    s = jax.lax.dot_general(
        q_ref[0], k_ref[0], (((1,), (1,)), ((), ())),
        preferred_element_type=jnp.float32,
    )
    s = s - jnp.max(s, axis=1, keepdims=True)
    p = jnp.exp(s) * m_ref[0]
    denom = jnp.maximum(jnp.sum(p, axis=1, keepdims=True), 1e-30)
    att = p / denom
    out_ref[0] = jax.lax.dot_general(
        att, v_ref[0], (((1,), (0,)), ((), ())),
        preferred_element_type=jnp.float32,
    )


def kernel(x, row_index, col_index, att_bias, Wq, bq, Wk, bk, Wv, bv, Wo, bo):
    n, d = x.shape
    h = att_bias.shape[0]
    dk = d // h
    nnz = row_index.shape[0]
    br = 128
    nb = n // br

    scale = 1.0 / math.sqrt(dk)
    wqkv = jnp.concatenate([Wq.T * scale, Wk.T, Wv.T], axis=1)
    bqkv = jnp.concatenate([bq * scale, bk, bv]).reshape(1, 3 * d)

    bmax = jnp.max(att_bias)
    mvals = jnp.exp(att_bias - bmax)
    m = jnp.zeros((h, n, n), jnp.float32).at[:, row_index, col_index].add(mvals)

    qkv = pl.pallas_call(
        _qkv_proj_body,
        grid=(nb,),
        in_specs=[
            pl.BlockSpec((br, d), lambda i: (i, 0)),
            pl.BlockSpec((d, 3 * d), lambda i: (0, 0)),
            pl.BlockSpec((1, 3 * d), lambda i: (0, 0)),
        ],
        out_specs=pl.BlockSpec((br, 3 * d), lambda i: (i, 0)),
        out_shape=jax.ShapeDtypeStruct((n, 3 * d), jnp.float32),
    )(x, wqkv, bqkv)

    qkv3 = qkv.reshape(n, 3 * h, dk).transpose(1, 0, 2)

    y3 = pl.pallas_call(
        _attn_body,
        grid=(h, nb),
        in_specs=[
            pl.BlockSpec((1, br, dk), lambda hh, i: (hh, i, 0)),
            pl.BlockSpec((1, n, dk), lambda hh, i: (h + hh, 0, 0)),
            pl.BlockSpec((1, n, dk), lambda hh, i: (2 * h + hh, 0, 0)),
            pl.BlockSpec((1, br, n), lambda hh, i: (hh, i, 0)),
        ],
        out_specs=pl.BlockSpec((1, br, dk), lambda hh, i: (hh, i, 0)),
        out_shape=jax.ShapeDtypeStruct((h, n, dk), jnp.float32),
    )(qkv3, qkv3, qkv3, m)

    y = y3.transpose(1, 0, 2).reshape(n, d)

    out = pl.pallas_call(
        _qkv_proj_body,
        grid=(nb,),
        in_specs=[
            pl.BlockSpec((br, d), lambda i: (i, 0)),
            pl.BlockSpec((d, d), lambda i: (0, 0)),
            pl.BlockSpec((1, d), lambda i: (0, 0)),
        ],
        out_specs=pl.BlockSpec((br, d), lambda i: (i, 0)),
        out_shape=jax.ShapeDtypeStruct((n, d), jnp.float32),
    )(y, Wo.T, bo.reshape(1, d))
    return out


# speedup vs baseline: 14.8963x; 2.2896x over previous
import functools
import math

import jax
import jax.numpy as jnp
from jax import lax
from jax.experimental import pallas as pl
from jax.experimental.pallas import tpu as pltpu
from jax.experimental.pallas import tpu_sc as plsc

_N = 2048
_H = 16
_NNZ = 65536
_CH = 1024
_RB = 32
_NB_ROWS = _N // _RB
_PAD = _CH


def _read_bound(bv_ref, j):
    w = (j // 16) * 16
    vec = bv_ref[pl.ds(w, 16)]
    lane = lax.iota(jnp.int32, 16)
    return jnp.sum(jnp.where(lane == (j - w), vec, 0))


def _mask_body(row_hbm, col_hbm, mval_hbm, bounds_hbm, out_hbm,
               bounds_v, canvas, row_c, col_c, val_c):
    wid = lax.axis_index("s") * 2 + lax.axis_index("c")
    pltpu.sync_copy(bounds_hbm, bounds_v)
    zero16 = jnp.zeros((16,), jnp.float32)
    lane = lax.iota(jnp.int32, 16)

    def _zrow(r, _):
        def _zcol(j, _):
            canvas[r, pl.ds(j * 16, 16)] = zero16
            return 0
        return lax.fori_loop(0, _N // 16, _zcol, 0)
    lax.fori_loop(0, _RB, _zrow, 0)

    def _chunk_pass(h, lo_a, n_chunks, p0, add):
        def _one_chunk(ci, _):
            base = lo_a + ci * _CH
            pltpu.sync_copy(row_hbm.at[pl.ds(base, _CH)], row_c)
            pltpu.sync_copy(col_hbm.at[pl.ds(base, _CH)], col_c)
            if add:
                pltpu.sync_copy(
                    mval_hbm.at[pl.ds(h * (_NNZ + _PAD) + base, _CH)], val_c)
            for s in range(_CH // 16):
                rv = row_c[pl.ds(s * 16, 16)]
                cv = col_c[pl.ds(s * 16, 16)]
                local = rv - p0
                m = (local >= 0) & (local < _RB)
                if add:
                    vv = val_c[pl.ds(s * 16, 16)]
                    plsc.addupdate_scatter(canvas, [local, cv], vv, mask=m)
                else:
                    plsc.store_scatter(canvas, [local, cv], zero16, mask=m)
            return 0
        lax.fori_loop(0, n_chunks, _one_chunk, 0)

    for half in range(2):
        j = wid * 2 + half
        lo = _read_bound(bounds_v, j)
        hi = _read_bound(bounds_v, j + 1)
        lo_a = (lo // 8) * 8
        n_chunks = (hi - lo_a + _CH - 1) // _CH
        p0 = j * _RB

        def _per_head(h, _):
            _chunk_pass(h, lo_a, n_chunks, p0, True)
            pltpu.sync_copy(canvas, out_hbm.at[h, pl.ds(p0, _RB)])
            _chunk_pass(h, lo_a, n_chunks, p0, False)
            return 0
        lax.fori_loop(0, _H, _per_head, 0)


def _build_mask_sc(row_index, col_index, mvals):
    n, h, nnz = _N, _H, _NNZ
    rowp = jnp.concatenate([row_index.astype(jnp.int32),
                            jnp.full((_PAD,), n, jnp.int32)])
    colp = jnp.concatenate([col_index.astype(jnp.int32),
                            jnp.zeros((_PAD,), jnp.int32)])
    mvalsp = jnp.pad(mvals, ((0, 0), (0, _PAD))).reshape(-1)
    bounds = jnp.searchsorted(
        row_index.astype(jnp.int32), jnp.arange(0, n + 1, _RB, dtype=jnp.int32)
    ).astype(jnp.int32)
    boundsp = jnp.pad(bounds, (0, 7))

    mesh = plsc.VectorSubcoreMesh(core_axis_name="c", subcore_axis_name="s")
    f = functools.partial(
        pl.kernel, mesh=mesh,
        compiler_params=pltpu.CompilerParams(needs_layout_passes=False),
        out_type=jax.ShapeDtypeStruct((h, n, n), jnp.float32),
        scratch_types=[
            pltpu.VMEM((72,), jnp.int32),
            pltpu.VMEM((_RB, _N), jnp.float32),
            pltpu.VMEM((_CH,), jnp.int32),
            pltpu.VMEM((_CH,), jnp.int32),
            pltpu.VMEM((_CH,), jnp.float32),
        ],
    )(_mask_body)
    return f(rowp, colp, mvalsp, boundsp)


def _qkv_proj_body(x_ref, w_ref, b_ref, out_ref):
    out_ref[...] = (
        jax.lax.dot_general(
            x_ref[...], w_ref[...], (((1,), (0,)), ((), ())),
            preferred_element_type=jnp.float32,
        )
        + b_ref[...]
    )


def _attn_body(q_ref, k_ref, v_ref, m_ref, out_ref):
    s = jax.lax.dot_general(
        q_ref[0], k_ref[0], (((1,), (1,)), ((), ())),
        preferred_element_type=jnp.float32,
    )
    s = s - jnp.max(s, axis=1, keepdims=True)
    p = jnp.exp(s) * m_ref[0]
    denom = jnp.maximum(jnp.sum(p, axis=1, keepdims=True), 1e-30)
    att = p / denom
    out_ref[0] = jax.lax.dot_general(
        att, v_ref[0], (((1,), (0,)), ((), ())),
        preferred_element_type=jnp.float32,
    )


def kernel(x, row_index, col_index, att_bias, Wq, bq, Wk, bk, Wv, bv, Wo, bo):
    n, d = x.shape
    h = att_bias.shape[0]
    dk = d // h
    nnz = row_index.shape[0]
    br = 128
    nb = n // br

    scale = 1.0 / math.sqrt(dk)
    wqkv = jnp.concatenate([Wq.T * scale, Wk.T, Wv.T], axis=1)
    bqkv = jnp.concatenate([bq * scale, bk, bv]).reshape(1, 3 * d)

    bmax = jnp.max(att_bias)
    mvals = jnp.exp(att_bias - bmax)
    m = _build_mask_sc(row_index, col_index, mvals)

    qkv = pl.pallas_call(
        _qkv_proj_body,
        grid=(nb,),
        in_specs=[
            pl.BlockSpec((br, d), lambda i: (i, 0)),
            pl.BlockSpec((d, 3 * d), lambda i: (0, 0)),
            pl.BlockSpec((1, 3 * d), lambda i: (0, 0)),
        ],
        out_specs=pl.BlockSpec((br, 3 * d), lambda i: (i, 0)),
        out_shape=jax.ShapeDtypeStruct((n, 3 * d), jnp.float32),
    )(x, wqkv, bqkv)

    qkv3 = qkv.reshape(n, 3 * h, dk).transpose(1, 0, 2)

    y3 = pl.pallas_call(
        _attn_body,
        grid=(h, nb),
        in_specs=[
            pl.BlockSpec((1, br, dk), lambda hh, i: (hh, i, 0)),
            pl.BlockSpec((1, n, dk), lambda hh, i: (h + hh, 0, 0)),
            pl.BlockSpec((1, n, dk), lambda hh, i: (2 * h + hh, 0, 0)),
            pl.BlockSpec((1, br, n), lambda hh, i: (hh, i, 0)),
        ],
        out_specs=pl.BlockSpec((1, br, dk), lambda hh, i: (hh, i, 0)),
        out_shape=jax.ShapeDtypeStruct((h, n, dk), jnp.float32),
    )(qkv3, qkv3, qkv3, m)

    y = y3.transpose(1, 0, 2).reshape(n, d)

    out = pl.pallas_call(
        _qkv_proj_body,
        grid=(nb,),
        in_specs=[
            pl.BlockSpec((br, d), lambda i: (i, 0)),
            pl.BlockSpec((d, d), lambda i: (0, 0)),
            pl.BlockSpec((1, d), lambda i: (0, 0)),
        ],
        out_specs=pl.BlockSpec((br, d), lambda i: (i, 0)),
        out_shape=jax.ShapeDtypeStruct((n, d), jnp.float32),
    )(y, Wo.T, bo.reshape(1, d))
    return out


# speedup vs baseline: 17.4311x; 1.1702x over previous
import functools
import math

import jax
import jax.numpy as jnp
from jax import lax
from jax.experimental import pallas as pl
from jax.experimental.pallas import tpu as pltpu
from jax.experimental.pallas import tpu_sc as plsc

_N = 2048
_H = 16
_NNZ = 65536
_CH = 512
_RU = 8
_HG = 4
_NU = _N // _RU
_PAD = _CH


def _read_bound(bv_ref, j):
    w = (j // 16) * 16
    vec = bv_ref[pl.ds(w, 16)]
    lane = lax.iota(jnp.int32, 16)
    return jnp.sum(jnp.where(lane == (j - w), vec, 0))


def _mask_body(row_hbm, col_hbm, mval_hbm, bounds_hbm, out_hbm,
               bounds_v, canvas, row_c, col_c, val_c, lsem, wsem):
    wid = lax.axis_index("s") * 2 + lax.axis_index("c")
    pltpu.sync_copy(bounds_hbm, bounds_v)
    zero16 = jnp.zeros((16,), jnp.float32)
    lane = lax.iota(jnp.int32, 16)

    def _zrow(r, _):
        def _zcol(jj, _):
            canvas[r // _RU, r % _RU, pl.ds(jj * 16, 16)] = zero16
            return 0
        return lax.fori_loop(0, _N // 16, _zcol, 0)
    lax.fori_loop(0, _HG * _RU, _zrow, 0)

    def _scan_chunks(g, lo, hi, p0, add):
        lo_a = (lo // 8) * 8
        n_chunks = (hi - lo_a + _CH - 1) // _CH

        def _one_chunk(ci, _):
            base = lo_a + ci * _CH
            c1 = pltpu.async_copy(row_hbm.at[pl.ds(base, _CH)], row_c, lsem)
            c2 = pltpu.async_copy(col_hbm.at[pl.ds(base, _CH)], col_c, lsem)
            if add:
                c3 = pltpu.async_copy(
                    mval_hbm.at[pl.ds(base * _H, _CH * _H)], val_c, lsem)
                c3.wait()
            c1.wait()
            c2.wait()
            s_lo = jnp.maximum(lo - base, 0) // 16
            s_hi = (jnp.minimum(hi - base, _CH) + 15) // 16

            def _one_vec(s, _):
                e = base + s * 16 + lane
                m = (e >= lo) & (e < hi)
                rv = row_c[pl.ds(s * 16, 16)]
                cv = col_c[pl.ds(s * 16, 16)]
                lrow = rv - p0
                vbase = (s * 16 + lane) * _H + g * _HG
                for hl in range(_HG):
                    hv = jnp.full((16,), hl, jnp.int32)
                    if add:
                        vv = plsc.load_gather(val_c, [vbase + hl])
                        plsc.addupdate_scatter(
                            canvas, [hv, lrow, cv], vv, mask=m)
                    else:
                        plsc.store_scatter(
                            canvas, [hv, lrow, cv], zero16, mask=m)
                return 0
            lax.fori_loop(s_lo, s_hi, _one_vec, 0)
            return 0
        lax.fori_loop(0, n_chunks, _one_chunk, 0)

    def _unit(t, _):
        u = wid * 8 + t // _HG
        g = t % _HG
        lo = _read_bound(bounds_v, u)
        hi = _read_bound(bounds_v, u + 1)
        p0 = u * _RU
        _scan_chunks(g, lo, hi, p0, True)
        ws = []
        for hl in range(_HG):
            ws.append(pltpu.async_copy(
                canvas.at[hl],
                out_hbm.at[g * _HG + hl, pl.ds(p0, _RU)], wsem))
        for w in ws:
            w.wait()
        _scan_chunks(g, lo, hi, p0, False)
        return 0
    lax.fori_loop(0, 32, _unit, 0)


def _build_mask_sc(row_index, col_index, mvals):
    n, h = _N, _H
    rowp = jnp.concatenate([row_index.astype(jnp.int32),
                            jnp.full((_PAD,), n, jnp.int32)])
    colp = jnp.concatenate([col_index.astype(jnp.int32),
                            jnp.zeros((_PAD,), jnp.int32)])
    mvalsp = jnp.pad(mvals.T, ((0, _PAD), (0, 0))).reshape(-1)
    bounds = jnp.searchsorted(
        row_index.astype(jnp.int32), jnp.arange(0, n + 1, _RU, dtype=jnp.int32)
    ).astype(jnp.int32)
    boundsp = jnp.pad(bounds, (0, 15))

    mesh = plsc.VectorSubcoreMesh(core_axis_name="c", subcore_axis_name="s")
    f = functools.partial(
        pl.kernel, mesh=mesh,
        compiler_params=pltpu.CompilerParams(needs_layout_passes=False),
        out_type=jax.ShapeDtypeStruct((h, n, n), jnp.float32),
        scratch_types=[
            pltpu.VMEM((272,), jnp.int32),
            pltpu.VMEM((_HG, _RU, _N), jnp.float32),
            pltpu.VMEM((_CH,), jnp.int32),
            pltpu.VMEM((_CH,), jnp.int32),
            pltpu.VMEM((_CH * _H,), jnp.float32),
            pltpu.SemaphoreType.DMA,
            pltpu.SemaphoreType.DMA,
        ],
    )(_mask_body)
    return f(rowp, colp, mvalsp, boundsp)


def _qkv_proj_body(x_ref, w_ref, b_ref, out_ref):
    out_ref[...] = (
        jax.lax.dot_general(
            x_ref[...], w_ref[...], (((1,), (0,)), ((), ())),
            preferred_element_type=jnp.float32,
        )
        + b_ref[...]
    )


def _attn_body(q_ref, k_ref, v_ref, m_ref, out_ref):
    s = jax.lax.dot_general(
        q_ref[0], k_ref[0], (((1,), (1,)), ((), ())),
        preferred_element_type=jnp.float32,
    )
    s = s - jnp.max(s, axis=1, keepdims=True)
    p = jnp.exp(s) * m_ref[0]
    denom = jnp.maximum(jnp.sum(p, axis=1, keepdims=True), 1e-30)
    att = p / denom
    out_ref[0] = jax.lax.dot_general(
        att, v_ref[0], (((1,), (0,)), ((), ())),
        preferred_element_type=jnp.float32,
    )


def kernel(x, row_index, col_index, att_bias, Wq, bq, Wk, bk, Wv, bv, Wo, bo):
    n, d = x.shape
    h = att_bias.shape[0]
    dk = d // h
    nnz = row_index.shape[0]
    br = 128
    nb = n // br

    scale = 1.0 / math.sqrt(dk)
    wqkv = jnp.concatenate([Wq.T * scale, Wk.T, Wv.T], axis=1)
    bqkv = jnp.concatenate([bq * scale, bk, bv]).reshape(1, 3 * d)

    bmax = jnp.max(att_bias)
    mvals = jnp.exp(att_bias - bmax)
    m = _build_mask_sc(row_index, col_index, mvals)

    qkv = pl.pallas_call(
        _qkv_proj_body,
        grid=(nb,),
        in_specs=[
            pl.BlockSpec((br, d), lambda i: (i, 0)),
            pl.BlockSpec((d, 3 * d), lambda i: (0, 0)),
            pl.BlockSpec((1, 3 * d), lambda i: (0, 0)),
        ],
        out_specs=pl.BlockSpec((br, 3 * d), lambda i: (i, 0)),
        out_shape=jax.ShapeDtypeStruct((n, 3 * d), jnp.float32),
    )(x, wqkv, bqkv)

    qkv3 = qkv.reshape(n, 3 * h, dk).transpose(1, 0, 2)

    y3 = pl.pallas_call(
        _attn_body,
        grid=(h, nb),
        in_specs=[
            pl.BlockSpec((1, br, dk), lambda hh, i: (hh, i, 0)),
            pl.BlockSpec((1, n, dk), lambda hh, i: (h + hh, 0, 0)),
            pl.BlockSpec((1, n, dk), lambda hh, i: (2 * h + hh, 0, 0)),
            pl.BlockSpec((1, br, n), lambda hh, i: (hh, i, 0)),
        ],
        out_specs=pl.BlockSpec((1, br, dk), lambda hh, i: (hh, i, 0)),
        out_shape=jax.ShapeDtypeStruct((h, n, dk), jnp.float32),
    )(qkv3, qkv3, qkv3, m)

    y = y3.transpose(1, 0, 2).reshape(n, d)

    out = pl.pallas_call(
        _qkv_proj_body,
        grid=(nb,),
        in_specs=[
            pl.BlockSpec((br, d), lambda i: (i, 0)),
            pl.BlockSpec((d, d), lambda i: (0, 0)),
            pl.BlockSpec((1, d), lambda i: (0, 0)),
        ],
        out_specs=pl.BlockSpec((br, d), lambda i: (i, 0)),
        out_shape=jax.ShapeDtypeStruct((n, d), jnp.float32),
    )(y, Wo.T, bo.reshape(1, d))
    return out


# speedup vs baseline: 18.5727x; 1.0655x over previous
import functools
import math

import jax
import jax.numpy as jnp
from jax import lax
from jax.experimental import pallas as pl
from jax.experimental.pallas import tpu as pltpu
from jax.experimental.pallas import tpu_sc as plsc

_N = 2048
_H = 16
_NNZ = 65536
_CH = 512
_RU = 8
_HG = 4
_NU = _N // _RU
_PAD = _CH


def _read_bound(bv_ref, j):
    w = (j // 16) * 16
    vec = bv_ref[pl.ds(w, 16)]
    lane = lax.iota(jnp.int32, 16)
    return jnp.sum(jnp.where(lane == (j - w), vec, 0))


def _mask_body(row_hbm, col_hbm, mval_hbm, bounds_hbm, out_hbm,
               bounds_v, canvas, row_c, col_c, val_c, lsem, wsem):
    wid = lax.axis_index("s") * 2 + lax.axis_index("c")
    pltpu.sync_copy(bounds_hbm, bounds_v)
    zero16 = jnp.zeros((16,), jnp.float32)
    lane = lax.iota(jnp.int32, 16)

    def _zrow(r, _):
        def _zcol(jj, _):
            canvas[r // _RU, r % _RU, pl.ds(jj * 16, 16)] = zero16
            return 0
        return lax.fori_loop(0, _N // 16, _zcol, 0)
    lax.fori_loop(0, _HG * _RU, _zrow, 0)

    def _scan_chunks(g, lo, hi, p0, add):
        lo_a = (lo // 8) * 8
        n_chunks = (hi - lo_a + _CH - 1) // _CH

        def _one_chunk(ci, _):
            base = lo_a + ci * _CH
            c1 = pltpu.async_copy(row_hbm.at[pl.ds(base, _CH)], row_c, lsem)
            c2 = pltpu.async_copy(col_hbm.at[pl.ds(base, _CH)], col_c, lsem)
            if add:
                c3 = pltpu.async_copy(
                    mval_hbm.at[pl.ds(base * _H, _CH * _H)], val_c, lsem)
                c3.wait()
            c1.wait()
            c2.wait()
            s_lo = jnp.maximum(lo - base, 0) // 16
            s_hi = (jnp.minimum(hi - base, _CH) + 15) // 16

            def _one_vec(s, _):
                e = base + s * 16 + lane
                m = (e >= lo) & (e < hi)
                rv = row_c[pl.ds(s * 16, 16)]
                cv = col_c[pl.ds(s * 16, 16)]
                lrow = rv - p0
                vbase = (s * 16 + lane) * _H + g * _HG
                for hl in range(_HG):
                    hv = jnp.full((16,), hl, jnp.int32)
                    if add:
                        vv = plsc.load_gather(val_c, [vbase + hl])
                        plsc.addupdate_scatter(
                            canvas, [hv, lrow, cv], vv, mask=m)
                    else:
                        plsc.store_scatter(
                            canvas, [hv, lrow, cv], zero16, mask=m)
                return 0
            lax.fori_loop(s_lo, s_hi, _one_vec, 0)
            return 0
        lax.fori_loop(0, n_chunks, _one_chunk, 0)

    def _unit(t, _):
        u = wid * 8 + t // _HG
        g = t % _HG
        lo = _read_bound(bounds_v, u)
        hi = _read_bound(bounds_v, u + 1)
        p0 = u * _RU
        _scan_chunks(g, lo, hi, p0, True)
        ws = []
        for hl in range(_HG):
            ws.append(pltpu.async_copy(
                canvas.at[hl],
                out_hbm.at[g * _HG + hl, pl.ds(p0, _RU)], wsem))
        for w in ws:
            w.wait()
        _scan_chunks(g, lo, hi, p0, False)
        return 0
    lax.fori_loop(0, 32, _unit, 0)


def _build_mask_sc(row_index, col_index, mvals):
    n, h = _N, _H
    rowp = jnp.concatenate([row_index.astype(jnp.int32),
                            jnp.full((_PAD,), n, jnp.int32)])
    colp = jnp.concatenate([col_index.astype(jnp.int32),
                            jnp.zeros((_PAD,), jnp.int32)])
    mvalsp = jnp.pad(mvals.T, ((0, _PAD), (0, 0))).reshape(-1)
    bounds = jnp.searchsorted(
        row_index.astype(jnp.int32), jnp.arange(0, n + 1, _RU, dtype=jnp.int32)
    ).astype(jnp.int32)
    boundsp = jnp.pad(bounds, (0, 15))

    mesh = plsc.VectorSubcoreMesh(core_axis_name="c", subcore_axis_name="s")
    f = functools.partial(
        pl.kernel, mesh=mesh,
        compiler_params=pltpu.CompilerParams(needs_layout_passes=False),
        out_type=jax.ShapeDtypeStruct((h, n, n), jnp.float32),
        scratch_types=[
            pltpu.VMEM((272,), jnp.int32),
            pltpu.VMEM((_HG, _RU, _N), jnp.float32),
            pltpu.VMEM((_CH,), jnp.int32),
            pltpu.VMEM((_CH,), jnp.int32),
            pltpu.VMEM((_CH * _H,), jnp.float32),
            pltpu.SemaphoreType.DMA,
            pltpu.SemaphoreType.DMA,
        ],
    )(_mask_body)
    return f(rowp, colp, mvalsp, boundsp)


def _qkv_proj_body(x_ref, w_ref, b_ref, out_ref):
    acc = jax.lax.dot_general(
        x_ref[...].astype(jnp.bfloat16), w_ref[...], (((1,), (0,)), ((), ())),
        preferred_element_type=jnp.float32,
    ) + b_ref[...]
    out_ref[...] = acc.astype(jnp.bfloat16)


def _out_proj_body(x_ref, w_ref, b_ref, out_ref):
    out_ref[...] = (
        jax.lax.dot_general(
            x_ref[...], w_ref[...], (((1,), (0,)), ((), ())),
            preferred_element_type=jnp.float32,
        )
        + b_ref[...]
    )


def _attn_body(q_ref, k_ref, v_ref, m_ref, out_ref):
    s = jax.lax.dot_general(
        q_ref[0], k_ref[0], (((1,), (1,)), ((), ())),
        preferred_element_type=jnp.float32,
    )
    s = s - jnp.max(s, axis=1, keepdims=True)
    p = jnp.exp(s) * m_ref[0]
    denom = jnp.maximum(jnp.sum(p, axis=1, keepdims=True), 1e-30)
    num = jax.lax.dot_general(
        p.astype(jnp.bfloat16), v_ref[0], (((1,), (0,)), ((), ())),
        preferred_element_type=jnp.float32,
    )
    out_ref[0] = num / denom


def kernel(x, row_index, col_index, att_bias, Wq, bq, Wk, bk, Wv, bv, Wo, bo):
    n, d = x.shape
    h = att_bias.shape[0]
    dk = d // h
    nnz = row_index.shape[0]
    br = 128
    nb = n // br

    scale = 1.0 / math.sqrt(dk)
    wqkv = jnp.concatenate([Wq.T * scale, Wk.T, Wv.T], axis=1).astype(
        jnp.bfloat16)
    bqkv = jnp.concatenate([bq * scale, bk, bv]).reshape(1, 3 * d)

    bmax = jnp.max(att_bias)
    mvals = jnp.exp(att_bias - bmax)
    m = _build_mask_sc(row_index, col_index, mvals)

    qkv = pl.pallas_call(
        _qkv_proj_body,
        grid=(nb,),
        in_specs=[
            pl.BlockSpec((br, d), lambda i: (i, 0)),
            pl.BlockSpec((d, 3 * d), lambda i: (0, 0)),
            pl.BlockSpec((1, 3 * d), lambda i: (0, 0)),
        ],
        out_specs=pl.BlockSpec((br, 3 * d), lambda i: (i, 0)),
        out_shape=jax.ShapeDtypeStruct((n, 3 * d), jnp.bfloat16),
    )(x, wqkv, bqkv)

    qkv3 = qkv.reshape(n, 3 * h, dk).transpose(1, 0, 2)

    y3 = pl.pallas_call(
        _attn_body,
        grid=(h, nb),
        in_specs=[
            pl.BlockSpec((1, br, dk), lambda hh, i: (hh, i, 0)),
            pl.BlockSpec((1, n, dk), lambda hh, i: (h + hh, 0, 0)),
            pl.BlockSpec((1, n, dk), lambda hh, i: (2 * h + hh, 0, 0)),
            pl.BlockSpec((1, br, n), lambda hh, i: (hh, i, 0)),
        ],
        out_specs=pl.BlockSpec((1, br, dk), lambda hh, i: (hh, i, 0)),
        out_shape=jax.ShapeDtypeStruct((h, n, dk), jnp.float32),
    )(qkv3, qkv3, qkv3, m)

    y = y3.transpose(1, 0, 2).reshape(n, d)

    out = pl.pallas_call(
        _out_proj_body,
        grid=(nb,),
        in_specs=[
            pl.BlockSpec((br, d), lambda i: (i, 0)),
            pl.BlockSpec((d, d), lambda i: (0, 0)),
            pl.BlockSpec((1, d), lambda i: (0, 0)),
        ],
        out_specs=pl.BlockSpec((br, d), lambda i: (i, 0)),
        out_shape=jax.ShapeDtypeStruct((n, d), jnp.float32),
    )(y, Wo.T, bo.reshape(1, d))
    return out


# speedup vs baseline: 22.0354x; 1.1864x over previous
import functools
import math

import jax
import jax.numpy as jnp
from jax import lax
from jax.experimental import pallas as pl
from jax.experimental.pallas import tpu as pltpu
from jax.experimental.pallas import tpu_sc as plsc

_N = 2048
_H = 16
_NNZ = 65536
_CH = 512
_RU = 8
_HG = 4
_NU = _N // _RU
_PAD = _CH
_NNZP = _NNZ + _PAD


def _read_bound(bv_ref, j):
    w = (j // 16) * 16
    vec = bv_ref[pl.ds(w, 16)]
    lane = lax.iota(jnp.int32, 16)
    return jnp.sum(jnp.where(lane == (j - w), vec, 0))


def _mask_body(row_hbm, col_hbm, mval_hbm, bounds_hbm, out_hbm,
               bounds_v, canvas, row_c, col_c, val_c, lsem, wsem):
    wid = lax.axis_index("s") * 2 + lax.axis_index("c")
    pltpu.sync_copy(bounds_hbm, bounds_v)
    zero16 = jnp.zeros((16,), jnp.float32)
    lane = lax.iota(jnp.int32, 16)

    def _zrow(r, _):
        def _zcol(jj, _):
            canvas[r // _RU, r % _RU, pl.ds(jj * 16, 16)] = zero16
            return 0
        return lax.fori_loop(0, _N // 16, _zcol, 0)
    lax.fori_loop(0, _HG * _RU, _zrow, 0)

    def _scan_chunks(g, lo, hi, p0, add):
        lo_a = (lo // 8) * 8
        n_chunks = (hi - lo_a + _CH - 1) // _CH

        def _one_chunk(ci, _):
            base = lo_a + ci * _CH
            cs = [pltpu.async_copy(row_hbm.at[pl.ds(base, _CH)], row_c, lsem),
                  pltpu.async_copy(col_hbm.at[pl.ds(base, _CH)], col_c, lsem)]
            if add:
                for hl in range(_HG):
                    cs.append(pltpu.async_copy(
                        mval_hbm.at[pl.ds((g * _HG + hl) * _NNZP + base, _CH)],
                        val_c.at[pl.ds(hl * _CH, _CH)], lsem))
            for c in cs:
                c.wait()
            s_lo = jnp.maximum(lo - base, 0) // 16
            s_hi = (jnp.minimum(hi - base, _CH) + 15) // 16

            def _one_vec(s, _):
                e = base + s * 16 + lane
                m = (e >= lo) & (e < hi)
                rv = row_c[pl.ds(s * 16, 16)]
                cv = col_c[pl.ds(s * 16, 16)]
                lrow = rv - p0
                ebase = s * 16 + lane
                for hl in range(_HG):
                    hv = jnp.full((16,), hl, jnp.int32)
                    if add:
                        vv = plsc.load_gather(val_c, [hl * _CH + ebase])
                        plsc.addupdate_scatter(
                            canvas, [hv, lrow, cv], vv, mask=m)
                    else:
                        plsc.store_scatter(
                            canvas, [hv, lrow, cv], zero16, mask=m)
                return 0
            lax.fori_loop(s_lo, s_hi, _one_vec, 0)
            return 0
        lax.fori_loop(0, n_chunks, _one_chunk, 0)

    def _unit(t, _):
        u = wid * 8 + t // _HG
        g = t % _HG
        lo = _read_bound(bounds_v, u)
        hi = _read_bound(bounds_v, u + 1)
        p0 = u * _RU
        _scan_chunks(g, lo, hi, p0, True)
        ws = []
        for hl in range(_HG):
            ws.append(pltpu.async_copy(
                canvas.at[hl],
                out_hbm.at[g * _HG + hl, pl.ds(p0, _RU)], wsem))
        for w in ws:
            w.wait()
        _scan_chunks(g, lo, hi, p0, False)
        return 0
    lax.fori_loop(0, 32, _unit, 0)


def _build_mask_sc(row_index, col_index, mvals):
    n, h = _N, _H
    rowp = jnp.concatenate([row_index.astype(jnp.int32),
                            jnp.full((_PAD,), n, jnp.int32)])
    colp = jnp.concatenate([col_index.astype(jnp.int32),
                            jnp.zeros((_PAD,), jnp.int32)])
    mvalsp = jnp.pad(mvals, ((0, 0), (0, _PAD))).reshape(-1)
    edges = jnp.arange(0, n + 1, _RU, dtype=jnp.int32)
    bounds = jnp.sum(
        row_index.astype(jnp.int32)[None, :] < edges[:, None],
        axis=1, dtype=jnp.int32)
    boundsp = jnp.pad(bounds, (0, 15))

    mesh = plsc.VectorSubcoreMesh(core_axis_name="c", subcore_axis_name="s")
    f = functools.partial(
        pl.kernel, mesh=mesh,
        compiler_params=pltpu.CompilerParams(needs_layout_passes=False),
        out_type=jax.ShapeDtypeStruct((h, n, n), jnp.float32),
        scratch_types=[
            pltpu.VMEM((272,), jnp.int32),
            pltpu.VMEM((_HG, _RU, _N), jnp.float32),
            pltpu.VMEM((_CH,), jnp.int32),
            pltpu.VMEM((_CH,), jnp.int32),
            pltpu.VMEM((_CH * _HG,), jnp.float32),
            pltpu.SemaphoreType.DMA,
            pltpu.SemaphoreType.DMA,
        ],
    )(_mask_body)
    return f(rowp, colp, mvalsp, boundsp)


def _qkv_proj_body(x_ref, w_ref, b_ref, out_ref):
    acc = jax.lax.dot_general(
        x_ref[...].astype(jnp.bfloat16), w_ref[...], (((1,), (0,)), ((), ())),
        preferred_element_type=jnp.float32,
    ) + b_ref[...]
    out_ref[...] = acc.astype(jnp.bfloat16)


def _out_proj_body(x_ref, w_ref, b_ref, out_ref):
    out_ref[...] = (
        jax.lax.dot_general(
            x_ref[...], w_ref[...], (((1,), (0,)), ((), ())),
            preferred_element_type=jnp.float32,
        )
        + b_ref[...]
    )


def _attn_body(q_ref, k_ref, v_ref, m_ref, out_ref):
    s = jax.lax.dot_general(
        q_ref[0], k_ref[0], (((1,), (1,)), ((), ())),
        preferred_element_type=jnp.float32,
    )
    s = s - jnp.max(s, axis=1, keepdims=True)
    p = jnp.exp(s) * m_ref[0]
    denom = jnp.maximum(jnp.sum(p, axis=1, keepdims=True), 1e-30)
    num = jax.lax.dot_general(
        p.astype(jnp.bfloat16), v_ref[0], (((1,), (0,)), ((), ())),
        preferred_element_type=jnp.float32,
    )
    out_ref[0] = num / denom


def kernel(x, row_index, col_index, att_bias, Wq, bq, Wk, bk, Wv, bv, Wo, bo):
    n, d = x.shape
    h = att_bias.shape[0]
    dk = d // h
    nnz = row_index.shape[0]
    br = 128
    nb = n // br

    scale = 1.0 / math.sqrt(dk)
    wqkv = jnp.concatenate([Wq.T * scale, Wk.T, Wv.T], axis=1).astype(
        jnp.bfloat16)
    bqkv = jnp.concatenate([bq * scale, bk, bv]).reshape(1, 3 * d)

    bmax = jnp.max(att_bias)
    mvals = jnp.exp(att_bias - bmax)
    m = _build_mask_sc(row_index, col_index, mvals)

    qkv = pl.pallas_call(
        _qkv_proj_body,
        grid=(nb,),
        in_specs=[
            pl.BlockSpec((br, d), lambda i: (i, 0)),
            pl.BlockSpec((d, 3 * d), lambda i: (0, 0)),
            pl.BlockSpec((1, 3 * d), lambda i: (0, 0)),
        ],
        out_specs=pl.BlockSpec((br, 3 * d), lambda i: (i, 0)),
        out_shape=jax.ShapeDtypeStruct((n, 3 * d), jnp.bfloat16),
    )(x, wqkv, bqkv)

    qkv3 = qkv.reshape(n, 3 * h, dk).transpose(1, 0, 2)

    y3 = pl.pallas_call(
        _attn_body,
        grid=(h, nb),
        in_specs=[
            pl.BlockSpec((1, br, dk), lambda hh, i: (hh, i, 0)),
            pl.BlockSpec((1, n, dk), lambda hh, i: (h + hh, 0, 0)),
            pl.BlockSpec((1, n, dk), lambda hh, i: (2 * h + hh, 0, 0)),
            pl.BlockSpec((1, br, n), lambda hh, i: (hh, i, 0)),
        ],
        out_specs=pl.BlockSpec((1, br, dk), lambda hh, i: (hh, i, 0)),
        out_shape=jax.ShapeDtypeStruct((h, n, dk), jnp.float32),
    )(qkv3, qkv3, qkv3, m)

    y = y3.transpose(1, 0, 2).reshape(n, d)

    out = pl.pallas_call(
        _out_proj_body,
        grid=(nb,),
        in_specs=[
            pl.BlockSpec((br, d), lambda i: (i, 0)),
            pl.BlockSpec((d, d), lambda i: (0, 0)),
            pl.BlockSpec((1, d), lambda i: (0, 0)),
        ],
        out_specs=pl.BlockSpec((br, d), lambda i: (i, 0)),
        out_shape=jax.ShapeDtypeStruct((n, d), jnp.float32),
    )(y, Wo.T, bo.reshape(1, d))
    return out


# speedup vs baseline: 24.3786x; 1.1063x over previous
import functools
import math

import jax
import jax.numpy as jnp
from jax import lax
from jax.experimental import pallas as pl
from jax.experimental.pallas import tpu as pltpu
from jax.experimental.pallas import tpu_sc as plsc

_N = 2048
_H = 16
_NNZ = 65536
_CH = 512
_RU = 8
_HG = 4
_NU = _N // _RU
_PAD = _CH
_NNZP = _NNZ + _PAD


def _read_bound(bv_ref, j):
    w = (j // 16) * 16
    vec = bv_ref[pl.ds(w, 16)]
    lane = lax.iota(jnp.int32, 16)
    return jnp.sum(jnp.where(lane == (j - w), vec, 0))


def _mask_body(row_hbm, col_hbm, mval_hbm, bounds_hbm, out_hbm,
               bounds_v, canvas, row_c, col_c, val_c, lsem, wsem):
    wid = lax.axis_index("s") * 2 + lax.axis_index("c")
    pltpu.sync_copy(bounds_hbm, bounds_v)
    zero16 = jnp.zeros((16,), jnp.float32)
    lane = lax.iota(jnp.int32, 16)

    def _zrow(r, _):
        def _zcol(jj, _):
            canvas[r // _RU, r % _RU, pl.ds(jj * 16, 16)] = zero16
            return 0
        return lax.fori_loop(0, _N // 16, _zcol, 0)
    lax.fori_loop(0, _HG * _RU, _zrow, 0)

    def _scan_chunks(g, lo, hi, p0, add):
        lo_a = (lo // 8) * 8
        n_chunks = (hi - lo_a + _CH - 1) // _CH

        def _one_chunk(ci, _):
            base = lo_a + ci * _CH
            cs = [pltpu.async_copy(row_hbm.at[pl.ds(base, _CH)], row_c, lsem),
                  pltpu.async_copy(col_hbm.at[pl.ds(base, _CH)], col_c, lsem)]
            if add:
                for hl in range(_HG):
                    cs.append(pltpu.async_copy(
                        mval_hbm.at[pl.ds((g * _HG + hl) * _NNZP + base, _CH)],
                        val_c.at[pl.ds(hl * _CH, _CH)], lsem))
            for c in cs:
                c.wait()
            s_lo = jnp.maximum(lo - base, 0) // 16
            s_hi = (jnp.minimum(hi - base, _CH) + 15) // 16

            def _one_vec(s, _):
                e = base + s * 16 + lane
                m = (e >= lo) & (e < hi)
                rv = row_c[pl.ds(s * 16, 16)]
                cv = col_c[pl.ds(s * 16, 16)]
                lrow = rv - p0
                ebase = s * 16 + lane
                for hl in range(_HG):
                    hv = jnp.full((16,), hl, jnp.int32)
                    if add:
                        vv = plsc.load_gather(val_c, [hl * _CH + ebase])
                        plsc.addupdate_scatter(
                            canvas, [hv, lrow, cv], vv, mask=m)
                    else:
                        plsc.store_scatter(
                            canvas, [hv, lrow, cv], zero16, mask=m)
                return 0
            lax.fori_loop(s_lo, s_hi, _one_vec, 0)
            return 0
        lax.fori_loop(0, n_chunks, _one_chunk, 0)

    def _unit(t, _):
        u = wid * 8 + t
        g = 0
        lo = _read_bound(bounds_v, u)
        hi = _read_bound(bounds_v, u + 1)
        p0 = u * _RU
        _scan_chunks(g, lo, hi, p0, True)
        ws = []
        for hl in range(_HG):
            ws.append(pltpu.async_copy(
                canvas.at[hl],
                out_hbm.at[hl, pl.ds(p0, _RU)], wsem))
        for w in ws:
            w.wait()
        _scan_chunks(g, lo, hi, p0, False)
        return 0
    lax.fori_loop(0, 8, _unit, 0)


def _prep_mask_inputs(row_index, col_index):
    n = _N
    rowp = jnp.concatenate([row_index.astype(jnp.int32),
                            jnp.full((_PAD,), n, jnp.int32)])
    colp = jnp.concatenate([col_index.astype(jnp.int32),
                            jnp.zeros((_PAD,), jnp.int32)])
    edges = jnp.arange(0, n + 1, _RU, dtype=jnp.int32)
    bounds = jnp.sum(
        row_index.astype(jnp.int32)[None, :] < edges[:, None],
        axis=1, dtype=jnp.int32)
    boundsp = jnp.pad(bounds, (0, 15))
    return rowp, colp, boundsp


def _build_mask_sc(rowp, colp, boundsp, mvals_g):
    mvalsp = jnp.pad(mvals_g, ((0, 0), (0, _PAD))).reshape(-1)
    mesh = plsc.VectorSubcoreMesh(core_axis_name="c", subcore_axis_name="s")
    f = functools.partial(
        pl.kernel, mesh=mesh,
        compiler_params=pltpu.CompilerParams(needs_layout_passes=False),
        out_type=jax.ShapeDtypeStruct((_HG, _N, _N), jnp.float32),
        scratch_types=[
            pltpu.VMEM((272,), jnp.int32),
            pltpu.VMEM((_HG, _RU, _N), jnp.float32),
            pltpu.VMEM((_CH,), jnp.int32),
            pltpu.VMEM((_CH,), jnp.int32),
            pltpu.VMEM((_CH * _HG,), jnp.float32),
            pltpu.SemaphoreType.DMA,
            pltpu.SemaphoreType.DMA,
        ],
    )(_mask_body)
    return f(rowp, colp, mvalsp, boundsp)


def _qkv_proj_body(x_ref, w_ref, b_ref, out_ref):
    acc = jax.lax.dot_general(
        x_ref[...].astype(jnp.bfloat16), w_ref[...], (((1,), (0,)), ((), ())),
        preferred_element_type=jnp.float32,
    ) + b_ref[...]
    out_ref[...] = acc.astype(jnp.bfloat16)


def _out_proj_body(x_ref, w_ref, b_ref, out_ref):
    out_ref[...] = (
        jax.lax.dot_general(
            x_ref[...], w_ref[...], (((1,), (0,)), ((), ())),
            preferred_element_type=jnp.float32,
        )
        + b_ref[...]
    )


def _attn_body(q_ref, k_ref, v_ref, m_ref, out_ref):
    s = jax.lax.dot_general(
        q_ref[0], k_ref[0], (((1,), (1,)), ((), ())),
        preferred_element_type=jnp.float32,
    )
    s = s - jnp.max(s, axis=1, keepdims=True)
    p = jnp.exp(s) * m_ref[0]
    denom = jnp.maximum(jnp.sum(p, axis=1, keepdims=True), 1e-30)
    num = jax.lax.dot_general(
        p.astype(jnp.bfloat16), v_ref[0], (((1,), (0,)), ((), ())),
        preferred_element_type=jnp.float32,
    )
    out_ref[0] = num / denom


def kernel(x, row_index, col_index, att_bias, Wq, bq, Wk, bk, Wv, bv, Wo, bo):
    n, d = x.shape
    h = att_bias.shape[0]
    dk = d // h
    nnz = row_index.shape[0]
    br = 128
    nb = n // br

    scale = 1.0 / math.sqrt(dk)
    wqkv = jnp.concatenate([Wq.T * scale, Wk.T, Wv.T], axis=1).astype(
        jnp.bfloat16)
    bqkv = jnp.concatenate([bq * scale, bk, bv]).reshape(1, 3 * d)

    bmax = jnp.max(att_bias)
    mvals = jnp.exp(att_bias - bmax)
    rowp, colp, boundsp = _prep_mask_inputs(row_index, col_index)
    ms = [_build_mask_sc(rowp, colp, boundsp, mvals[g * _HG:(g + 1) * _HG])
          for g in range(h // _HG)]

    qkv = pl.pallas_call(
        _qkv_proj_body,
        grid=(nb,),
        in_specs=[
            pl.BlockSpec((br, d), lambda i: (i, 0)),
            pl.BlockSpec((d, 3 * d), lambda i: (0, 0)),
            pl.BlockSpec((1, 3 * d), lambda i: (0, 0)),
        ],
        out_specs=pl.BlockSpec((br, 3 * d), lambda i: (i, 0)),
        out_shape=jax.ShapeDtypeStruct((n, 3 * d), jnp.bfloat16),
    )(x, wqkv, bqkv)

    qkv3 = qkv.reshape(n, 3 * h, dk).transpose(1, 0, 2)

    y3s = []
    for g in range(h // _HG):
        y3s.append(pl.pallas_call(
            _attn_body,
            grid=(_HG, nb),
            in_specs=[
                pl.BlockSpec((1, br, dk),
                             lambda hh, i, g=g: (g * _HG + hh, i, 0)),
                pl.BlockSpec((1, n, dk),
                             lambda hh, i, g=g: (h + g * _HG + hh, 0, 0)),
                pl.BlockSpec((1, n, dk),
                             lambda hh, i, g=g: (2 * h + g * _HG + hh, 0, 0)),
                pl.BlockSpec((1, br, n), lambda hh, i: (hh, i, 0)),
            ],
            out_specs=pl.BlockSpec((1, br, dk), lambda hh, i: (hh, i, 0)),
            out_shape=jax.ShapeDtypeStruct((_HG, n, dk), jnp.float32),
        )(qkv3, qkv3, qkv3, ms[g]))

    y3 = jnp.concatenate(y3s, axis=0)
    y = y3.transpose(1, 0, 2).reshape(n, d)

    out = pl.pallas_call(
        _out_proj_body,
        grid=(nb,),
        in_specs=[
            pl.BlockSpec((br, d), lambda i: (i, 0)),
            pl.BlockSpec((d, d), lambda i: (0, 0)),
            pl.BlockSpec((1, d), lambda i: (0, 0)),
        ],
        out_specs=pl.BlockSpec((br, d), lambda i: (i, 0)),
        out_shape=jax.ShapeDtypeStruct((n, d), jnp.float32),
    )(y, Wo.T, bo.reshape(1, d))
    return out


# speedup vs baseline: 25.8017x; 1.0584x over previous
import functools
import math

import jax
import jax.numpy as jnp
from jax import lax
from jax.experimental import pallas as pl
from jax.experimental.pallas import tpu as pltpu
from jax.experimental.pallas import tpu_sc as plsc

_N = 2048
_H = 16
_NNZ = 65536
_CH = 512
_RU = 8
_HG = 4
_NU = _N // _RU
_PAD = _CH
_NNZP = _NNZ + _PAD


def _read_bound(bv_ref, j):
    w = (j // 16) * 16
    vec = bv_ref[pl.ds(w, 16)]
    lane = lax.iota(jnp.int32, 16)
    return jnp.sum(jnp.where(lane == (j - w), vec, 0))


def _mask_body(row_hbm, col_hbm, mval_hbm, bounds_hbm, out_hbm,
               bounds_v, canvas, row_c, col_c, val_c, lsem, wsem):
    wid = lax.axis_index("s") * 2 + lax.axis_index("c")
    pltpu.sync_copy(bounds_hbm, bounds_v)
    zero16 = jnp.zeros((16,), jnp.float32)
    lane = lax.iota(jnp.int32, 16)

    def _zrow(r, _):
        def _zcol(jj, _):
            canvas[r // _RU, r % _RU, pl.ds(jj * 16, 16)] = zero16
            return 0
        return lax.fori_loop(0, _N // 16, _zcol, 0)
    lax.fori_loop(0, _HG * _RU, _zrow, 0)

    def _scan_chunks(g, lo, hi, p0, add):
        lo_a = (lo // 8) * 8
        n_chunks = (hi - lo_a + _CH - 1) // _CH

        def _one_chunk(ci, _):
            base_u = lo_a + ci * _CH
            base = jnp.minimum(base_u, _NNZ - _CH)
            estart = jnp.maximum(lo, base_u)
            eend = jnp.minimum(hi, base_u + _CH)
            cs = [pltpu.async_copy(row_hbm.at[pl.ds(base, _CH)], row_c, lsem),
                  pltpu.async_copy(col_hbm.at[pl.ds(base, _CH)], col_c, lsem)]
            if add:
                for hl in range(_HG):
                    cs.append(pltpu.async_copy(
                        mval_hbm.at[pl.ds(hl * _NNZ + base, _CH)],
                        val_c.at[pl.ds(hl * _CH, _CH)], lsem))
            for c in cs:
                c.wait()
            s_lo = (estart - base) // 16
            s_hi = (eend - base + 15) // 16

            def _one_vec(s, _):
                e = base + s * 16 + lane
                m = (e >= estart) & (e < eend)
                rv = row_c[pl.ds(s * 16, 16)]
                cv = col_c[pl.ds(s * 16, 16)]
                lrow = rv - p0
                ebase = s * 16 + lane
                for hl in range(_HG):
                    hv = jnp.full((16,), hl, jnp.int32)
                    if add:
                        vv = plsc.load_gather(val_c, [hl * _CH + ebase])
                        plsc.addupdate_scatter(
                            canvas, [hv, lrow, cv], vv, mask=m)
                    else:
                        plsc.store_scatter(
                            canvas, [hv, lrow, cv], zero16, mask=m)
                return 0
            lax.fori_loop(s_lo, s_hi, _one_vec, 0)
            return 0
        lax.fori_loop(0, n_chunks, _one_chunk, 0)

    def _unit(t, _):
        u = wid * 8 + t
        g = 0
        lo = _read_bound(bounds_v, u)
        hi = _read_bound(bounds_v, u + 1)
        p0 = u * _RU
        _scan_chunks(g, lo, hi, p0, True)
        ws = []
        for hl in range(_HG):
            ws.append(pltpu.async_copy(
                canvas.at[hl],
                out_hbm.at[hl, pl.ds(p0, _RU)], wsem))
        for w in ws:
            w.wait()
        _scan_chunks(g, lo, hi, p0, False)
        return 0
    lax.fori_loop(0, 8, _unit, 0)


def _prep_mask_inputs(row_index, col_index):
    edges = jnp.minimum(jnp.arange(272, dtype=jnp.int32) * _RU, _N)
    bounds = jnp.sum(
        row_index.astype(jnp.int32)[None, :] < edges[:, None],
        axis=1, dtype=jnp.int32)
    return row_index.astype(jnp.int32), col_index.astype(jnp.int32), bounds


def _build_mask_sc(rowp, colp, boundsp, mvals_g):
    mvalsp = mvals_g.reshape(-1)
    mesh = plsc.VectorSubcoreMesh(core_axis_name="c", subcore_axis_name="s")
    f = functools.partial(
        pl.kernel, mesh=mesh,
        compiler_params=pltpu.CompilerParams(needs_layout_passes=False),
        out_type=jax.ShapeDtypeStruct((_HG, _N, _N), jnp.float32),
        scratch_types=[
            pltpu.VMEM((272,), jnp.int32),
            pltpu.VMEM((_HG, _RU, _N), jnp.float32),
            pltpu.VMEM((_CH,), jnp.int32),
            pltpu.VMEM((_CH,), jnp.int32),
            pltpu.VMEM((_CH * _HG,), jnp.float32),
            pltpu.SemaphoreType.DMA,
            pltpu.SemaphoreType.DMA,
        ],
    )(_mask_body)
    return f(rowp, colp, mvalsp, boundsp)


def _qkv_proj_body(x_ref, w_ref, b_ref, out_ref):
    acc = jax.lax.dot_general(
        x_ref[...].astype(jnp.bfloat16), w_ref[...], (((1,), (0,)), ((), ())),
        preferred_element_type=jnp.float32,
    ) + b_ref[...]
    out_ref[...] = acc.astype(jnp.bfloat16)


def _out_proj_body(x_ref, w_ref, b_ref, out_ref):
    out_ref[...] = (
        jax.lax.dot_general(
            x_ref[...], w_ref[...], (((1,), (0,)), ((), ())),
            preferred_element_type=jnp.float32,
        )
        + b_ref[...]
    )


def _attn_body(q_ref, k_ref, v_ref, m_ref, out_ref):
    s = jax.lax.dot_general(
        q_ref[0], k_ref[0], (((1,), (1,)), ((), ())),
        preferred_element_type=jnp.float32,
    )
    p = jnp.exp(s) * m_ref[0]
    denom = jnp.maximum(jnp.sum(p, axis=1, keepdims=True), 1e-30)
    num = jax.lax.dot_general(
        p.astype(jnp.bfloat16), v_ref[0], (((1,), (0,)), ((), ())),
        preferred_element_type=jnp.float32,
    )
    out_ref[0] = num / denom


def kernel(x, row_index, col_index, att_bias, Wq, bq, Wk, bk, Wv, bv, Wo, bo):
    n, d = x.shape
    h = att_bias.shape[0]
    dk = d // h
    nnz = row_index.shape[0]
    br = 128
    nb = n // br

    scale = 1.0 / math.sqrt(dk)
    wqkv = jnp.concatenate([Wq.T * scale, Wk.T, Wv.T], axis=1).astype(
        jnp.bfloat16)
    bqkv = jnp.concatenate([bq * scale, bk, bv]).reshape(1, 3 * d)

    bmax = jnp.max(att_bias)
    mvals = jnp.exp(att_bias - bmax)
    rowp, colp, boundsp = _prep_mask_inputs(row_index, col_index)
    ms = [_build_mask_sc(rowp, colp, boundsp, mvals[g * _HG:(g + 1) * _HG])
          for g in range(h // _HG)]

    qkv = pl.pallas_call(
        _qkv_proj_body,
        grid=(nb,),
        in_specs=[
            pl.BlockSpec((br, d), lambda i: (i, 0)),
            pl.BlockSpec((d, 3 * d), lambda i: (0, 0)),
            pl.BlockSpec((1, 3 * d), lambda i: (0, 0)),
        ],
        out_specs=pl.BlockSpec((br, 3 * d), lambda i: (i, 0)),
        out_shape=jax.ShapeDtypeStruct((n, 3 * d), jnp.bfloat16),
    )(x, wqkv, bqkv)

    qkv3 = qkv.reshape(n, 3 * h, dk).transpose(1, 0, 2)

    y3s = []
    for g in range(h // _HG):
        y3s.append(pl.pallas_call(
            _attn_body,
            grid=(_HG, nb),
            in_specs=[
                pl.BlockSpec((1, br, dk),
                             lambda hh, i, g=g: (g * _HG + hh, i, 0)),
                pl.BlockSpec((1, n, dk),
                             lambda hh, i, g=g: (h + g * _HG + hh, 0, 0)),
                pl.BlockSpec((1, n, dk),
                             lambda hh, i, g=g: (2 * h + g * _HG + hh, 0, 0)),
                pl.BlockSpec((1, br, n), lambda hh, i: (hh, i, 0)),
            ],
            out_specs=pl.BlockSpec((1, br, dk), lambda hh, i: (hh, i, 0)),
            out_shape=jax.ShapeDtypeStruct((_HG, n, dk), jnp.float32),
        )(qkv3, qkv3, qkv3, ms[g]))

    y3 = jnp.concatenate(y3s, axis=0)
    y = y3.transpose(1, 0, 2).reshape(n, d)

    out = pl.pallas_call(
        _out_proj_body,
        grid=(nb,),
        in_specs=[
            pl.BlockSpec((br, d), lambda i: (i, 0)),
            pl.BlockSpec((d, d), lambda i: (0, 0)),
            pl.BlockSpec((1, d), lambda i: (0, 0)),
        ],
        out_specs=pl.BlockSpec((br, d), lambda i: (i, 0)),
        out_shape=jax.ShapeDtypeStruct((n, d), jnp.float32),
    )(y, Wo.T, bo.reshape(1, d))
    return out


# speedup vs baseline: 26.5465x; 1.0289x over previous
import functools
import math

import jax
import jax.numpy as jnp
from jax import lax
from jax.experimental import pallas as pl
from jax.experimental.pallas import tpu as pltpu
from jax.experimental.pallas import tpu_sc as plsc

_N = 2048
_H = 16
_NNZ = 65536
_CH = 512
_RU = 8
_HG = 4
_NU = _N // _RU
_PAD = _CH
_NNZP = _NNZ + _PAD


def _read_bound(bv_ref, j):
    w = (j // 16) * 16
    vec = bv_ref[pl.ds(w, 16)]
    lane = lax.iota(jnp.int32, 16)
    return jnp.sum(jnp.where(lane == (j - w), vec, 0))


def _mask_body(row_hbm, col_hbm, mval_hbm, bounds_hbm, out_hbm,
               bounds_v, canvas, row_c, col_c, val_c, lsem, wsem):
    wid = lax.axis_index("s") * 2 + lax.axis_index("c")
    pltpu.sync_copy(bounds_hbm, bounds_v)
    zero16 = jnp.zeros((16,), jnp.float32)
    lane = lax.iota(jnp.int32, 16)

    def _zrow(r, _):
        def _zcol(jo, _):
            for k in range(8):
                canvas[r // _RU, r % _RU, pl.ds((jo * 8 + k) * 16, 16)] = zero16
            return 0
        return lax.fori_loop(0, _N // 128, _zcol, 0)
    lax.fori_loop(0, _HG * _RU, _zrow, 0)

    def _scan_chunks(g, lo, hi, p0, add):
        lo_a = (lo // 8) * 8
        n_chunks = (hi - lo_a + _CH - 1) // _CH

        def _one_chunk(ci, _):
            base_u = lo_a + ci * _CH
            base = jnp.minimum(base_u, _NNZ - _CH)
            estart = jnp.maximum(lo, base_u)
            eend = jnp.minimum(hi, base_u + _CH)
            cs = [pltpu.async_copy(row_hbm.at[pl.ds(base, _CH)], row_c, lsem),
                  pltpu.async_copy(col_hbm.at[pl.ds(base, _CH)], col_c, lsem)]
            if add:
                for hl in range(_HG):
                    cs.append(pltpu.async_copy(
                        mval_hbm.at[pl.ds(hl * _NNZ + base, _CH)],
                        val_c.at[pl.ds(hl * _CH, _CH)], lsem))
            for c in cs:
                c.wait()
            s_lo = (estart - base) // 16
            s_hi = (eend - base + 15) // 16

            def _one_vec(s, _):
                e = base + s * 16 + lane
                m = (e >= estart) & (e < eend)
                rv = row_c[pl.ds(s * 16, 16)]
                cv = col_c[pl.ds(s * 16, 16)]
                lrow = rv - p0
                ebase = s * 16 + lane
                for hl in range(_HG):
                    hv = jnp.full((16,), hl, jnp.int32)
                    if add:
                        vv = plsc.load_gather(val_c, [hl * _CH + ebase])
                        plsc.addupdate_scatter(
                            canvas, [hv, lrow, cv], vv, mask=m)
                    else:
                        plsc.store_scatter(
                            canvas, [hv, lrow, cv], zero16, mask=m)
                return 0
            lax.fori_loop(s_lo, s_hi, _one_vec, 0)
            return 0
        lax.fori_loop(0, n_chunks, _one_chunk, 0)

    def _unit(t, _):
        u = wid * 8 + t
        g = 0
        lo = _read_bound(bounds_v, u)
        hi = _read_bound(bounds_v, u + 1)
        p0 = u * _RU
        _scan_chunks(g, lo, hi, p0, True)
        ws = []
        for hl in range(_HG):
            ws.append(pltpu.async_copy(
                canvas.at[hl],
                out_hbm.at[hl, pl.ds(p0, _RU)], wsem))
        for w in ws:
            w.wait()
        _scan_chunks(g, lo, hi, p0, False)
        return 0
    lax.fori_loop(0, 8, _unit, 0)


def _prep_mask_inputs(row_index, col_index):
    edges = jnp.minimum(jnp.arange(272, dtype=jnp.int32) * _RU, _N)
    bounds = jnp.sum(
        row_index.astype(jnp.int32)[None, :] < edges[:, None],
        axis=1, dtype=jnp.int32)
    return row_index.astype(jnp.int32), col_index.astype(jnp.int32), bounds


def _build_mask_sc(rowp, colp, boundsp, mvals_g):
    mvalsp = mvals_g.reshape(-1)
    mesh = plsc.VectorSubcoreMesh(core_axis_name="c", subcore_axis_name="s")
    f = functools.partial(
        pl.kernel, mesh=mesh,
        compiler_params=pltpu.CompilerParams(needs_layout_passes=False),
        out_type=jax.ShapeDtypeStruct((_HG, _N, _N), jnp.float32),
        scratch_types=[
            pltpu.VMEM((272,), jnp.int32),
            pltpu.VMEM((_HG, _RU, _N), jnp.float32),
            pltpu.VMEM((_CH,), jnp.int32),
            pltpu.VMEM((_CH,), jnp.int32),
            pltpu.VMEM((_CH * _HG,), jnp.float32),
            pltpu.SemaphoreType.DMA,
            pltpu.SemaphoreType.DMA,
        ],
    )(_mask_body)
    return f(rowp, colp, mvalsp, boundsp)


def _qkv_proj_body(x_ref, w_ref, b_ref, out_ref):
    acc = jax.lax.dot_general(
        x_ref[...].astype(jnp.bfloat16), w_ref[...], (((1,), (0,)), ((), ())),
        preferred_element_type=jnp.float32,
    ) + b_ref[...]
    out_ref[...] = acc.astype(jnp.bfloat16)


def _out_proj_body(x_ref, w_ref, b_ref, out_ref):
    out_ref[...] = (
        jax.lax.dot_general(
            x_ref[...], w_ref[...], (((1,), (0,)), ((), ())),
            preferred_element_type=jnp.float32,
        )
        + b_ref[...]
    )


def _attn_body(q_ref, k_ref, v_ref, m_ref, out_ref):
    s = jax.lax.dot_general(
        q_ref[0], k_ref[0], (((1,), (1,)), ((), ())),
        preferred_element_type=jnp.float32,
    )
    p = jnp.exp(s) * m_ref[0]
    denom = jnp.maximum(jnp.sum(p, axis=1, keepdims=True), 1e-30)
    num = jax.lax.dot_general(
        p.astype(jnp.bfloat16), v_ref[0], (((1,), (0,)), ((), ())),
        preferred_element_type=jnp.float32,
    )
    out_ref[0] = num / denom


def kernel(x, row_index, col_index, att_bias, Wq, bq, Wk, bk, Wv, bv, Wo, bo):
    n, d = x.shape
    h = att_bias.shape[0]
    dk = d // h
    nnz = row_index.shape[0]
    br = 128
    nb = n // br

    scale = 1.0 / math.sqrt(dk)
    wqkv = jnp.concatenate([Wq.T * scale, Wk.T, Wv.T], axis=1).astype(
        jnp.bfloat16)
    bqkv = jnp.concatenate([bq * scale, bk, bv]).reshape(1, 3 * d)

    bmax = jnp.max(att_bias)
    mvals = jnp.exp(att_bias - bmax)
    rowp, colp, boundsp = _prep_mask_inputs(row_index, col_index)
    ms = [_build_mask_sc(rowp, colp, boundsp, mvals[g * _HG:(g + 1) * _HG])
          for g in range(h // _HG)]

    qkv = pl.pallas_call(
        _qkv_proj_body,
        grid=(nb,),
        in_specs=[
            pl.BlockSpec((br, d), lambda i: (i, 0)),
            pl.BlockSpec((d, 3 * d), lambda i: (0, 0)),
            pl.BlockSpec((1, 3 * d), lambda i: (0, 0)),
        ],
        out_specs=pl.BlockSpec((br, 3 * d), lambda i: (i, 0)),
        out_shape=jax.ShapeDtypeStruct((n, 3 * d), jnp.bfloat16),
    )(x, wqkv, bqkv)

    qkv3 = qkv.reshape(n, 3 * h, dk).transpose(1, 0, 2)

    y3s = []
    for g in range(h // _HG):
        y3s.append(pl.pallas_call(
            _attn_body,
            grid=(_HG, nb),
            in_specs=[
                pl.BlockSpec((1, br, dk),
                             lambda hh, i, g=g: (g * _HG + hh, i, 0)),
                pl.BlockSpec((1, n, dk),
                             lambda hh, i, g=g: (h + g * _HG + hh, 0, 0)),
                pl.BlockSpec((1, n, dk),
                             lambda hh, i, g=g: (2 * h + g * _HG + hh, 0, 0)),
                pl.BlockSpec((1, br, n), lambda hh, i: (hh, i, 0)),
            ],
            out_specs=pl.BlockSpec((1, br, dk), lambda hh, i: (hh, i, 0)),
            out_shape=jax.ShapeDtypeStruct((_HG, n, dk), jnp.float32),
        )(qkv3, qkv3, qkv3, ms[g]))

    y3 = jnp.concatenate(y3s, axis=0)
    y = y3.transpose(1, 0, 2).reshape(n, d)

    out = pl.pallas_call(
        _out_proj_body,
        grid=(nb,),
        in_specs=[
            pl.BlockSpec((br, d), lambda i: (i, 0)),
            pl.BlockSpec((d, d), lambda i: (0, 0)),
            pl.BlockSpec((1, d), lambda i: (0, 0)),
        ],
        out_specs=pl.BlockSpec((br, d), lambda i: (i, 0)),
        out_shape=jax.ShapeDtypeStruct((n, d), jnp.float32),
    )(y, Wo.T, bo.reshape(1, d))
    return out


# speedup vs baseline: 26.7210x; 1.0066x over previous
import functools
import math

import jax
import jax.numpy as jnp
from jax import lax
from jax.experimental import pallas as pl
from jax.experimental.pallas import tpu as pltpu
from jax.experimental.pallas import tpu_sc as plsc

_N = 2048
_H = 16
_NNZ = 65536
_CH = 512
_RU = 8
_HG = 4
_NU = _N // _RU
_PAD = _CH
_NNZP = _NNZ + _PAD


def _read_bound(bv_ref, j):
    w = (j // 16) * 16
    vec = bv_ref[pl.ds(w, 16)]
    lane = lax.iota(jnp.int32, 16)
    return jnp.sum(jnp.where(lane == (j - w), vec, 0))


def _mask_body(row_hbm, col_hbm, mval_hbm, bounds_hbm, out_hbm,
               bounds_v, canvas, row_c, col_c, val_c, lsem, wsem):
    wid = lax.axis_index("s") * 2 + lax.axis_index("c")
    pltpu.sync_copy(bounds_hbm, bounds_v)
    zero16 = jnp.zeros((16,), jnp.float32)
    lane = lax.iota(jnp.int32, 16)

    def _zrow(r, _):
        def _zcol(jo, _):
            for k in range(8):
                canvas[r // _RU, r % _RU, pl.ds((jo * 8 + k) * 16, 16)] = zero16
            return 0
        return lax.fori_loop(0, _N // 128, _zcol, 0)
    lax.fori_loop(0, _HG * _RU, _zrow, 0)

    def _scan_chunks(g, lo, hi, p0, add):
        lo_a = (lo // 8) * 8
        n_chunks = (hi - lo_a + _CH - 1) // _CH

        def _one_chunk(ci, _):
            base_u = lo_a + ci * _CH
            base = jnp.minimum(base_u, _NNZ - _CH)
            estart = jnp.maximum(lo, base_u)
            eend = jnp.minimum(hi, base_u + _CH)
            cs = [pltpu.async_copy(row_hbm.at[pl.ds(base, _CH)], row_c, lsem),
                  pltpu.async_copy(col_hbm.at[pl.ds(base, _CH)], col_c, lsem)]
            if add:
                for hl in range(_HG):
                    cs.append(pltpu.async_copy(
                        mval_hbm.at[pl.ds(hl * _NNZ + base, _CH)],
                        val_c.at[pl.ds(hl * _CH, _CH)], lsem))
            for c in cs:
                c.wait()
            s_lo = (estart - base) // 16
            s_hi = (eend - base + 15) // 16

            def _one_vec(s, _):
                e = base + s * 16 + lane
                m = (e >= estart) & (e < eend)
                rv = row_c[pl.ds(s * 16, 16)]
                cv = col_c[pl.ds(s * 16, 16)]
                lrow = rv - p0
                ebase = s * 16 + lane
                for hl in range(_HG):
                    hv = jnp.full((16,), hl, jnp.int32)
                    if add:
                        vv = plsc.load_gather(val_c, [hl * _CH + ebase])
                        plsc.addupdate_scatter(
                            canvas, [hv, lrow, cv], vv, mask=m)
                    else:
                        plsc.store_scatter(
                            canvas, [hv, lrow, cv], zero16, mask=m)
                return 0
            lax.fori_loop(s_lo, s_hi, _one_vec, 0)
            return 0
        lax.fori_loop(0, n_chunks, _one_chunk, 0)

    def _unit(t, _):
        u = wid * 8 + t
        g = 0
        lo = _read_bound(bounds_v, u)
        hi = _read_bound(bounds_v, u + 1)
        p0 = u * _RU
        _scan_chunks(g, lo, hi, p0, True)
        ws = []
        for hl in range(_HG):
            ws.append(pltpu.async_copy(
                canvas.at[hl],
                out_hbm.at[hl, pl.ds(p0, _RU)], wsem))
        for w in ws:
            w.wait()
        _scan_chunks(g, lo, hi, p0, False)
        return 0
    lax.fori_loop(0, 8, _unit, 0)


def _prep_mask_inputs(row_index, col_index):
    edges = jnp.minimum(jnp.arange(272, dtype=jnp.int32) * _RU, _N)
    bounds = jnp.sum(
        row_index.astype(jnp.int32)[None, :] < edges[:, None],
        axis=1, dtype=jnp.int32)
    return row_index.astype(jnp.int32), col_index.astype(jnp.int32), bounds


def _build_mask_sc(rowp, colp, boundsp, mvalsp):
    mesh = plsc.VectorSubcoreMesh(core_axis_name="c", subcore_axis_name="s")
    f = functools.partial(
        pl.kernel, mesh=mesh,
        compiler_params=pltpu.CompilerParams(needs_layout_passes=False),
        out_type=jax.ShapeDtypeStruct((_HG, _N, _N), jnp.float32),
        scratch_types=[
            pltpu.VMEM((272,), jnp.int32),
            pltpu.VMEM((_HG, _RU, _N), jnp.float32),
            pltpu.VMEM((_CH,), jnp.int32),
            pltpu.VMEM((_CH,), jnp.int32),
            pltpu.VMEM((_CH * _HG,), jnp.float32),
            pltpu.SemaphoreType.DMA,
            pltpu.SemaphoreType.DMA,
        ],
    )(_mask_body)
    return f(rowp, colp, mvalsp, boundsp)


def _qkv_proj_body(x_ref, w_ref, b_ref, out_ref):
    acc = jax.lax.dot_general(
        x_ref[...].astype(jnp.bfloat16), w_ref[...], (((1,), (0,)), ((), ())),
        preferred_element_type=jnp.float32,
    ) + b_ref[...]
    out_ref[...] = acc.astype(jnp.bfloat16)


def _out_proj_body(x_ref, w_ref, b_ref, out_ref):
    out_ref[...] = (
        jax.lax.dot_general(
            x_ref[...], w_ref[...], (((1,), (0,)), ((), ())),
            preferred_element_type=jnp.float32,
        )
        + b_ref[...]
    )


def _attn_body(q_ref, k_ref, v_ref, m_ref, out_ref):
    s = jax.lax.dot_general(
        q_ref[0], k_ref[0], (((1,), (1,)), ((), ())),
        preferred_element_type=jnp.float32,
    )
    p = jnp.exp(s) * m_ref[0]
    denom = jnp.maximum(jnp.sum(p, axis=1, keepdims=True), 1e-30)
    num = jax.lax.dot_general(
        p.astype(jnp.bfloat16), v_ref[0], (((1,), (0,)), ((), ())),
        preferred_element_type=jnp.float32,
    )
    out_ref[0] = num / denom


def kernel(x, row_index, col_index, att_bias, Wq, bq, Wk, bk, Wv, bv, Wo, bo):
    n, d = x.shape
    h = att_bias.shape[0]
    dk = d // h
    nnz = row_index.shape[0]
    br = 128
    nb = n // br

    scale = 1.0 / math.sqrt(dk)
    wqkv = jnp.concatenate([Wq.T * scale, Wk.T, Wv.T], axis=1).astype(
        jnp.bfloat16)
    bqkv = jnp.concatenate([bq * scale, bk, bv]).reshape(1, 3 * d)

    bmax = jnp.max(att_bias)
    rowp, colp, boundsp = _prep_mask_inputs(row_index, col_index)
    ms = [_build_mask_sc(
              rowp, colp, boundsp,
              jnp.exp(att_bias[g * _HG:(g + 1) * _HG] - bmax).reshape(-1))
          for g in range(h // _HG)]

    qkv = pl.pallas_call(
        _qkv_proj_body,
        grid=(nb,),
        in_specs=[
            pl.BlockSpec((br, d), lambda i: (i, 0)),
            pl.BlockSpec((d, 3 * d), lambda i: (0, 0)),
            pl.BlockSpec((1, 3 * d), lambda i: (0, 0)),
        ],
        out_specs=pl.BlockSpec((br, 3 * d), lambda i: (i, 0)),
        out_shape=jax.ShapeDtypeStruct((n, 3 * d), jnp.bfloat16),
    )(x, wqkv, bqkv)

    qkv3 = qkv.reshape(n, 3 * h, dk).transpose(1, 0, 2)

    y3s = []
    for g in range(h // _HG):
        y3s.append(pl.pallas_call(
            _attn_body,
            grid=(_HG, nb),
            in_specs=[
                pl.BlockSpec((1, br, dk),
                             lambda hh, i, g=g: (g * _HG + hh, i, 0)),
                pl.BlockSpec((1, n, dk),
                             lambda hh, i, g=g: (h + g * _HG + hh, 0, 0)),
                pl.BlockSpec((1, n, dk),
                             lambda hh, i, g=g: (2 * h + g * _HG + hh, 0, 0)),
                pl.BlockSpec((1, br, n), lambda hh, i: (hh, i, 0)),
            ],
            out_specs=pl.BlockSpec((1, br, dk), lambda hh, i: (hh, i, 0)),
            out_shape=jax.ShapeDtypeStruct((_HG, n, dk), jnp.float32),
        )(qkv3, qkv3, qkv3, ms[g]))

    y3 = jnp.concatenate(y3s, axis=0)
    y = y3.transpose(1, 0, 2).reshape(n, d)

    out = pl.pallas_call(
        _out_proj_body,
        grid=(nb,),
        in_specs=[
            pl.BlockSpec((br, d), lambda i: (i, 0)),
            pl.BlockSpec((d, d), lambda i: (0, 0)),
            pl.BlockSpec((1, d), lambda i: (0, 0)),
        ],
        out_specs=pl.BlockSpec((br, d), lambda i: (i, 0)),
        out_shape=jax.ShapeDtypeStruct((n, d), jnp.float32),
    )(y, Wo.T, bo.reshape(1, d))
    return out


# speedup vs baseline: 38.6891x; 1.4479x over previous
import functools
import math

import jax
import jax.numpy as jnp
from jax import lax
from jax.experimental import pallas as pl
from jax.experimental.pallas import tpu as pltpu
from jax.experimental.pallas import tpu_sc as plsc

_N = 2048
_H = 16
_NNZ = 65536
_CH = 512
_RU = 8
_HG = 4
_NU = _N // _RU
_PAD = _CH
_NNZP = _NNZ + _PAD


def _read_bound(bv_ref, j):
    w = (j // 16) * 16
    vec = bv_ref[pl.ds(w, 16)]
    lane = lax.iota(jnp.int32, 16)
    return jnp.sum(jnp.where(lane == (j - w), vec, 0))


def _mask_body(row_hbm, col_hbm, mval_hbm, bounds_hbm, out_hbm,
               bounds_v, canvas, row_c, col_c, val_c, lsem, wsem):
    wid = lax.axis_index("s") * 2 + lax.axis_index("c")
    pltpu.sync_copy(bounds_hbm, bounds_v)
    zero16 = jnp.zeros((16,), jnp.float32)
    lane = lax.iota(jnp.int32, 16)

    def _zrow(r, _):
        def _zcol(jo, _):
            for k in range(8):
                canvas[r // _RU, r % _RU, pl.ds((jo * 8 + k) * 16, 16)] = zero16
            return 0
        return lax.fori_loop(0, _N // 128, _zcol, 0)
    lax.fori_loop(0, _HG * _RU, _zrow, 0)

    def _scan_chunks(g, lo, hi, p0, add):
        lo_a = (lo // 8) * 8
        n_chunks = (hi - lo_a + _CH - 1) // _CH

        def _one_chunk(ci, _):
            base_u = lo_a + ci * _CH
            base = jnp.minimum(base_u, _NNZ - _CH)
            estart = jnp.maximum(lo, base_u)
            eend = jnp.minimum(hi, base_u + _CH)
            cs = [pltpu.async_copy(row_hbm.at[pl.ds(base, _CH)], row_c, lsem),
                  pltpu.async_copy(col_hbm.at[pl.ds(base, _CH)], col_c, lsem)]
            if add:
                for hl in range(_HG):
                    cs.append(pltpu.async_copy(
                        mval_hbm.at[pl.ds(hl * _NNZ + base, _CH)],
                        val_c.at[pl.ds(hl * _CH, _CH)], lsem))
            for c in cs:
                c.wait()
            s_lo = (estart - base) // 16
            s_hi = (eend - base + 15) // 16

            def _one_vec(s, _):
                e = base + s * 16 + lane
                m = (e >= estart) & (e < eend)
                rv = row_c[pl.ds(s * 16, 16)]
                cv = col_c[pl.ds(s * 16, 16)]
                lrow = rv - p0
                ebase = s * 16 + lane
                for hl in range(_HG):
                    hv = jnp.full((16,), hl, jnp.int32)
                    if add:
                        vv = plsc.load_gather(val_c, [hl * _CH + ebase])
                        plsc.addupdate_scatter(
                            canvas, [hv, lrow, cv], vv, mask=m)
                    else:
                        plsc.store_scatter(
                            canvas, [hv, lrow, cv], zero16, mask=m)
                return 0
            lax.fori_loop(s_lo, s_hi, _one_vec, 0)
            return 0
        lax.fori_loop(0, n_chunks, _one_chunk, 0)

    def _unit(t, _):
        u = wid * 8 + t
        g = 0
        lo = _read_bound(bounds_v, u)
        hi = _read_bound(bounds_v, u + 1)
        p0 = u * _RU
        _scan_chunks(g, lo, hi, p0, True)
        ws = []
        for hl in range(_HG):
            ws.append(pltpu.async_copy(
                canvas.at[hl],
                out_hbm.at[hl, pl.ds(p0, _RU)], wsem))
        for w in ws:
            w.wait()
        _scan_chunks(g, lo, hi, p0, False)
        return 0
    lax.fori_loop(0, 8, _unit, 0)


def _prep_mask_inputs(row_index, col_index):
    edges = jnp.minimum(jnp.arange(272, dtype=jnp.int32) * _RU, _N)
    bounds = jnp.sum(
        row_index.astype(jnp.int32)[None, :] < edges[:, None],
        axis=1, dtype=jnp.int32)
    return row_index.astype(jnp.int32), col_index.astype(jnp.int32), bounds


def _build_mask_sc(rowp, colp, boundsp, mvalsp):
    mesh = plsc.VectorSubcoreMesh(core_axis_name="c", subcore_axis_name="s")
    f = functools.partial(
        pl.kernel, mesh=mesh,
        compiler_params=pltpu.CompilerParams(needs_layout_passes=False),
        out_type=jax.ShapeDtypeStruct((_HG, _N, _N), jnp.float32),
        scratch_types=[
            pltpu.VMEM((272,), jnp.int32),
            pltpu.VMEM((_HG, _RU, _N), jnp.float32),
            pltpu.VMEM((_CH,), jnp.int32),
            pltpu.VMEM((_CH,), jnp.int32),
            pltpu.VMEM((_CH * _HG,), jnp.float32),
            pltpu.SemaphoreType.DMA,
            pltpu.SemaphoreType.DMA,
        ],
    )(_mask_body)
    return f(rowp, colp, mvalsp, boundsp)


def _qkv_proj_body(x_ref, w_ref, b_ref, out_ref):
    acc = jax.lax.dot_general(
        x_ref[...].astype(jnp.bfloat16), w_ref[...], (((1,), (0,)), ((), ())),
        preferred_element_type=jnp.float32,
    ) + b_ref[...]
    out_ref[...] = acc.astype(jnp.bfloat16)


def _out_proj_body(x_ref, w_ref, b_ref, out_ref):
    out_ref[...] = (
        jax.lax.dot_general(
            x_ref[...], w_ref[...], (((1,), (0,)), ((), ())),
            preferred_element_type=jnp.float32,
        )
        + b_ref[...]
    )


def _attn_body(q_ref, k_ref, v_ref, m_ref, out_ref):
    dk = 64
    outs = []
    for hh in range(_HG):
        q = q_ref[:, hh * dk:(hh + 1) * dk]
        k = k_ref[:, hh * dk:(hh + 1) * dk]
        v = v_ref[:, hh * dk:(hh + 1) * dk]
        s = jax.lax.dot_general(
            q, k, (((1,), (1,)), ((), ())),
            preferred_element_type=jnp.float32,
        )
        p = jnp.exp(s) * m_ref[hh]
        denom = jnp.maximum(jnp.sum(p, axis=1, keepdims=True), 1e-30)
        num = jax.lax.dot_general(
            p.astype(jnp.bfloat16), v, (((1,), (0,)), ((), ())),
            preferred_element_type=jnp.float32,
        )
        outs.append(num / denom)
    out_ref[...] = jnp.concatenate(outs, axis=1)


def kernel(x, row_index, col_index, att_bias, Wq, bq, Wk, bk, Wv, bv, Wo, bo):
    n, d = x.shape
    h = att_bias.shape[0]
    dk = d // h
    nnz = row_index.shape[0]
    br = 128
    nb = n // br

    scale = 1.0 / math.sqrt(dk)
    wqkv = jnp.concatenate([Wq.T * scale, Wk.T, Wv.T], axis=1).astype(
        jnp.bfloat16)
    bqkv = jnp.concatenate([bq * scale, bk, bv]).reshape(1, 3 * d)

    bmax = jnp.max(att_bias)
    rowp, colp, boundsp = _prep_mask_inputs(row_index, col_index)
    ms = [_build_mask_sc(
              rowp, colp, boundsp,
              jnp.exp(att_bias[g * _HG:(g + 1) * _HG] - bmax).reshape(-1))
          for g in range(h // _HG)]

    qkv = pl.pallas_call(
        _qkv_proj_body,
        grid=(nb,),
        in_specs=[
            pl.BlockSpec((br, d), lambda i: (i, 0)),
            pl.BlockSpec((d, 3 * d), lambda i: (0, 0)),
            pl.BlockSpec((1, 3 * d), lambda i: (0, 0)),
        ],
        out_specs=pl.BlockSpec((br, 3 * d), lambda i: (i, 0)),
        out_shape=jax.ShapeDtypeStruct((n, 3 * d), jnp.bfloat16),
    )(x, wqkv, bqkv)

    gw = _HG * dk
    ng = h // _HG
    ygs = []
    for g in range(ng):
        ygs.append(pl.pallas_call(
            _attn_body,
            grid=(nb,),
            in_specs=[
                pl.BlockSpec((br, gw), lambda i, g=g: (i, g)),
                pl.BlockSpec((n, gw), lambda i, g=g: (0, ng + g)),
                pl.BlockSpec((n, gw), lambda i, g=g: (0, 2 * ng + g)),
                pl.BlockSpec((_HG, br, n), lambda i: (0, i, 0)),
            ],
            out_specs=pl.BlockSpec((br, gw), lambda i: (i, 0)),
            out_shape=jax.ShapeDtypeStruct((n, gw), jnp.float32),
        )(qkv, qkv, qkv, ms[g]))

    y = jnp.concatenate(ygs, axis=1)

    out = pl.pallas_call(
        _out_proj_body,
        grid=(nb,),
        in_specs=[
            pl.BlockSpec((br, d), lambda i: (i, 0)),
            pl.BlockSpec((d, d), lambda i: (0, 0)),
            pl.BlockSpec((1, d), lambda i: (0, 0)),
        ],
        out_specs=pl.BlockSpec((br, d), lambda i: (i, 0)),
        out_shape=jax.ShapeDtypeStruct((n, d), jnp.float32),
    )(y, Wo.T, bo.reshape(1, d))
    return out
